# baseline (speedup 1.0000x reference)
import functools

import jax
import jax.numpy as jnp
from jax import lax
from jax.experimental import pallas as pl
from jax.experimental.pallas import tpu as pltpu
from jax.experimental.pallas import tpu_sc as plsc

L, B, D, E = 8192, 2, 768, 8
N = L * B
TM = 512
TL = TM // B
NB = N // TM
TMM = 512
G = N // TMM + E
NPAD = G * TMM
NW = 32
RW = N // NW
CH = 128
NCH = RW // CH
D2 = D // 2


def _pack_bf16(x32):
    xb = x32.astype(jnp.bfloat16)
    hi = lax.bitcast_convert_type(xb[:, :D2], jnp.uint16).astype(jnp.int32)
    lo = lax.bitcast_convert_type(xb[:, D2:], jnp.uint16).astype(jnp.int32)
    return (hi << 16) | lo


def _unpack_bf16(xi):
    hi = lax.bitcast_convert_type((xi >> 16).astype(jnp.uint16), jnp.bfloat16)
    lo = lax.bitcast_convert_type((xi & 0xFFFF).astype(jnp.uint16),
                                  jnp.bfloat16)
    return jnp.concatenate([hi, lo], axis=1)


def _routing_body(alpha_ref, x_ref, wg_ref, bg_ref, tp_ref, wt_ref, bt_ref,
                  xlin_ref, dest_ref, te_ref, laux_ref,
                  psum_ref, cnt_ref, sel_s, rank_s):
    i = pl.program_id(0)
    alpha = alpha_ref[0, 0]
    x = x_ref[...].reshape(TM, D)
    xlin_ref[...] = _pack_bf16(x)

    task_logits = (jnp.dot(tp_ref[...], wt_ref[...],
                           preferred_element_type=jnp.float32)
                   + bt_ref[...])
    logits = ((1.0 - alpha)
              * (jnp.dot(x, wg_ref[...], preferred_element_type=jnp.float32)
                 + bg_ref[...])
              + alpha * task_logits)
    logits = jnp.where(jnp.isfinite(logits), logits, 0.0)

    mx = jnp.max(logits, axis=-1, keepdims=True)
    m = (logits == mx).astype(jnp.int32)

    def _shift(v, k):
        return jnp.concatenate(
            [jnp.zeros((TM, k), v.dtype), v[:, :E - k]], axis=1)

    p = m
    p = p | _shift(p, 1)
    p = p | _shift(p, 2)
    p = p | _shift(p, 4)
    onehot = (m & (1 - _shift(p, 1))).astype(jnp.float32)

    ones_col = jnp.ones((E, 1), jnp.float32)
    iota_col = lax.broadcasted_iota(jnp.int32, (E, 1), 0).astype(jnp.float32)
    sel = jnp.dot(onehot, iota_col,
                  preferred_element_type=jnp.float32)

    ex = jnp.exp(logits - mx)
    denom = jnp.dot(ex, ones_col, preferred_element_type=jnp.float32)
    probs = ex / denom

    @pl.when(i == 0)
    def _init():
        psum_ref[...] = jnp.zeros_like(psum_ref)
        cnt_ref[...] = jnp.zeros_like(cnt_ref)

    lt = (lax.broadcasted_iota(jnp.int32, (TM, TM), 0)
          > lax.broadcasted_iota(jnp.int32, (TM, TM), 1)).astype(jnp.float32)
    ltoh = jnp.dot(lt, onehot, preferred_element_type=jnp.float32)
    rank = jnp.dot(onehot * (cnt_ref[...] + ltoh), ones_col,
                   preferred_element_type=jnp.float32)
    sel_s[i, :, :] = sel.astype(jnp.int32).reshape(1, TM)
    rank_s[i, :, :] = rank.astype(jnp.int32).reshape(1, TM)

    ones_row = jnp.ones((1, TM), jnp.float32)
    psum_ref[...] += jnp.dot(ones_row, probs,
                             preferred_element_type=jnp.float32)
    cnt_ref[...] += jnp.dot(ones_row, onehot,
                            preferred_element_type=jnp.float32)

    @pl.when(i == NB - 1)
    def _fin():
        laux_ref[0, 0] = jnp.sum(psum_ref[...] * cnt_ref[...]) / (N * N)
        iota1 = lax.broadcasted_iota(jnp.int32, (1, E), 1)
        starts = []
        s = jnp.int32(0)
        for e in range(E):
            ce = jnp.sum(
                jnp.where(iota1 == e, cnt_ref[...], 0.0)).astype(jnp.int32)
            starts.append(s)
            s = s + ((ce + TMM - 1) // TMM) * TMM
        iota_g = lax.broadcasted_iota(jnp.int32, (1, 128), 1) * TMM
        te = jnp.zeros((1, 128), jnp.int32)
        for e in range(1, E):
            te = te + (iota_g >= starts[e]).astype(jnp.int32)
        te_ref[...] = te
        for j in range(NB):
            selj = sel_s[j, :, :]
            destj = rank_s[j, :, :]
            for e in range(1, E):
                destj = destj + jnp.where(selj == e, starts[e], 0)
            dest_ref[j, :, :] = destj


def _routing(alpha2, x3, Wg, bg2, tp2, Wt, bt2):
    return pl.pallas_call(
        _routing_body,
        grid=(NB,),
        in_specs=[
            pl.BlockSpec(memory_space=pltpu.SMEM),
            pl.BlockSpec((TL, B, D), lambda i: (i, 0, 0)),
            pl.BlockSpec((D, E), lambda i: (0, 0)),
            pl.BlockSpec((1, E), lambda i: (0, 0)),
            pl.BlockSpec((1, D), lambda i: (0, 0)),
            pl.BlockSpec((D, E), lambda i: (0, 0)),
            pl.BlockSpec((1, E), lambda i: (0, 0)),
        ],
        out_specs=[
            pl.BlockSpec((TM, D2), lambda i: (i, 0)),
            pl.BlockSpec((NB, 1, TM), lambda i: (0, 0, 0)),
            pl.BlockSpec((1, 128), lambda i: (0, 0)),
            pl.BlockSpec(memory_space=pltpu.SMEM),
        ],
        out_shape=[
            jax.ShapeDtypeStruct((N, D2), jnp.int32),
            jax.ShapeDtypeStruct((NB, 1, TM), jnp.int32),
            jax.ShapeDtypeStruct((1, 128), jnp.int32),
            jax.ShapeDtypeStruct((1, 1), jnp.float32),
        ],
        scratch_shapes=[
            pltpu.VMEM((1, E), jnp.float32),
            pltpu.VMEM((1, E), jnp.float32),
            pltpu.VMEM((NB, 1, TM), jnp.int32),
            pltpu.VMEM((NB, 1, TM), jnp.int32),
        ],
    )(alpha2, x3, Wg, bg2, tp2, Wt, bt2)


_SC_MESH = plsc.VectorSubcoreMesh(core_axis_name="c", subcore_axis_name="s")


@functools.partial(
    pl.kernel, mesh=_SC_MESH,
    out_type=jax.ShapeDtypeStruct((NPAD, D2), jnp.int32),
    scratch_types=[
        pltpu.VMEM((CH,), jnp.int32),
        pltpu.VMEM((CH, D2), jnp.int32),
        pltpu.SemaphoreType.DMA,
    ],
)
def _sc_scatter(x_hbm, dest_hbm, xs_hbm, dest_v, rows_v, sem):
    wid = lax.axis_index("s") * 2 + lax.axis_index("c")
    base = wid * RW
    for k in range(NCH):
        off = base + k * CH
        pltpu.sync_copy(dest_hbm.at[pl.ds(off, CH)], dest_v)
        pltpu.sync_copy(x_hbm.at[pl.ds(off, CH)], rows_v)
        pltpu.async_copy(rows_v, xs_hbm.at[dest_v], sem).wait()


def _gmm_body(te_ref, xs_ref, we_ref, be_ref, ys_ref, wcur_ref, bcur_ref):
    i = pl.program_id(0)
    e = te_ref[i]
    changed = jnp.logical_or(i == 0, te_ref[jnp.maximum(i - 1, 0)] != e)

    @pl.when(changed)
    def _stage_weights():
        wcur_ref[...] = we_ref[pl.ds(e, 1)][0]
        bcur_ref[...] = be_ref[pl.ds(e, 1)][0]

    x = _unpack_bf16(xs_ref[...])
    ys_ref[...] = _pack_bf16(
        jnp.dot(x, wcur_ref[...], preferred_element_type=jnp.float32)
        + bcur_ref[...])


def _gmm(te, xs, We, be):
    grid_spec = pltpu.PrefetchScalarGridSpec(
        num_scalar_prefetch=1,
        grid=(G,),
        in_specs=[
            pl.BlockSpec((TMM, D2), lambda i, te: (i, 0)),
            pl.BlockSpec((E, D, D), lambda i, te: (0, 0, 0)),
            pl.BlockSpec((E, 1, D), lambda i, te: (0, 0, 0)),
        ],
        out_specs=pl.BlockSpec((TMM, D2), lambda i, te: (i, 0)),
        scratch_shapes=[
            pltpu.VMEM((D, D), jnp.bfloat16),
            pltpu.VMEM((1, D), jnp.float32),
        ],
    )
    return pl.pallas_call(
        _gmm_body,
        grid_spec=grid_spec,
        out_shape=jax.ShapeDtypeStruct((NPAD, D2), jnp.int32),
    )(te, xs, We.astype(jnp.bfloat16), be.reshape(E, 1, D))


@functools.partial(
    pl.kernel, mesh=_SC_MESH,
    out_type=jax.ShapeDtypeStruct((N, D2), jnp.int32),
    scratch_types=[
        pltpu.VMEM((CH,), jnp.int32),
        pltpu.VMEM((CH, D2), jnp.int32),
        pltpu.SemaphoreType.DMA,
    ],
)
def _sc_gather(ys_hbm, dest_hbm, out_hbm, dest_v, rows_v, sem):
    wid = lax.axis_index("s") * 2 + lax.axis_index("c")
    base = wid * RW
    for k in range(NCH):
        off = base + k * CH
        pltpu.sync_copy(dest_hbm.at[pl.ds(off, CH)], dest_v)
        pltpu.async_copy(ys_hbm.at[dest_v], rows_v, sem).wait()
        pltpu.sync_copy(rows_v, out_hbm.at[pl.ds(off, CH)])


TMR = 2048


def _relayout_body(ylin_ref, out_ref):
    y = _unpack_bf16(ylin_ref[...]).astype(jnp.float32)
    out_ref[...] = y.reshape(TMR // B, B, D)


def _relayout(ylin):
    return pl.pallas_call(
        _relayout_body,
        grid=(N // TMR,),
        in_specs=[pl.BlockSpec((TMR, D2), lambda i: (i, 0))],
        out_specs=pl.BlockSpec((TMR // B, B, D), lambda i: (i, 0, 0)),
        out_shape=jax.ShapeDtypeStruct((L, B, D), jnp.float32),
    )(ylin)


@jax.jit
def kernel(inputs, task_param, alpha, Wg, bg, Wt, bt, We, be):
    alpha2 = jnp.asarray(alpha, jnp.float32).reshape(1, 1)
    xlin, dest3, te, laux = _routing(
        alpha2, inputs, Wg, bg.reshape(1, E), task_param.reshape(1, D), Wt,
        bt.reshape(1, E))
    dest = dest3.reshape(N)
    xs = _sc_scatter(xlin, dest)
    ys = _gmm(te.reshape(128)[:G], xs, We, be)
    out_lin = _sc_gather(ys, dest)
    return _relayout(out_lin), laux[0, 0]


# speedup vs baseline: 1.0769x; 1.0769x over previous
import functools

import jax
import jax.numpy as jnp
from jax import lax
from jax.experimental import pallas as pl
from jax.experimental.pallas import tpu as pltpu
from jax.experimental.pallas import tpu_sc as plsc

L, B, D, E = 8192, 2, 768, 8
N = L * B
TM = 512
TL = TM // B
NB = N // TM
TMM = 512
G = N // TMM + E
NPAD = G * TMM
NW = 32
RW = N // NW
CH = 128
NCH = RW // CH
D2 = D // 2


def _pack_bf16(x32):
    xb = x32.astype(jnp.bfloat16)
    hi = lax.bitcast_convert_type(xb[:, :D2], jnp.uint16).astype(jnp.int32)
    lo = lax.bitcast_convert_type(xb[:, D2:], jnp.uint16).astype(jnp.int32)
    return (hi << 16) | lo


def _unpack_bf16(xi):
    hi = lax.bitcast_convert_type((xi >> 16).astype(jnp.uint16), jnp.bfloat16)
    lo = lax.bitcast_convert_type((xi & 0xFFFF).astype(jnp.uint16),
                                  jnp.bfloat16)
    return jnp.concatenate([hi, lo], axis=1)


def _routing_body(alpha_ref, x_ref, wg_ref, bg_ref, tp_ref, wt_ref, bt_ref,
                  xlin_ref, dest_ref, te_ref, laux_ref,
                  psum_ref, cnt_ref, sel_s, rank_s):
    i = pl.program_id(0)
    alpha = alpha_ref[0, 0]
    x = x_ref[...].reshape(TM, D)
    xlin_ref[...] = _pack_bf16(x)

    task_logits = (jnp.dot(tp_ref[...], wt_ref[...],
                           preferred_element_type=jnp.float32)
                   + bt_ref[...])
    logits = ((1.0 - alpha)
              * (jnp.dot(x, wg_ref[...], preferred_element_type=jnp.float32)
                 + bg_ref[...])
              + alpha * task_logits)
    logits = jnp.where(jnp.isfinite(logits), logits, 0.0)

    mx = jnp.max(logits, axis=-1, keepdims=True)
    m = (logits == mx).astype(jnp.int32)

    def _shift(v, k):
        return jnp.concatenate(
            [jnp.zeros((TM, k), v.dtype), v[:, :E - k]], axis=1)

    p = m
    p = p | _shift(p, 1)
    p = p | _shift(p, 2)
    p = p | _shift(p, 4)
    onehot = (m & (1 - _shift(p, 1))).astype(jnp.float32)

    ones_col = jnp.ones((E, 1), jnp.float32)
    iota_col = lax.broadcasted_iota(jnp.int32, (E, 1), 0).astype(jnp.float32)
    sel = jnp.dot(onehot, iota_col,
                  preferred_element_type=jnp.float32)

    ex = jnp.exp(logits - mx)
    denom = jnp.dot(ex, ones_col, preferred_element_type=jnp.float32)
    probs = ex / denom

    @pl.when(i == 0)
    def _init():
        psum_ref[...] = jnp.zeros_like(psum_ref)
        cnt_ref[...] = jnp.zeros_like(cnt_ref)

    lt = (lax.broadcasted_iota(jnp.int32, (TM, TM), 0)
          > lax.broadcasted_iota(jnp.int32, (TM, TM), 1)).astype(jnp.float32)
    ltoh = jnp.dot(lt, onehot, preferred_element_type=jnp.float32)
    rank = jnp.dot(onehot * (cnt_ref[...] + ltoh), ones_col,
                   precision=lax.Precision.HIGHEST,
                   preferred_element_type=jnp.float32)
    sel_s[i, :, :] = sel.astype(jnp.int32).reshape(1, TM)
    rank_s[i, :, :] = rank.astype(jnp.int32).reshape(1, TM)

    ones_row = jnp.ones((1, TM), jnp.float32)
    psum_ref[...] += jnp.dot(ones_row, probs,
                             preferred_element_type=jnp.float32)
    cnt_ref[...] += jnp.dot(ones_row, onehot,
                            preferred_element_type=jnp.float32)

    @pl.when(i == NB - 1)
    def _fin():
        laux_ref[0, 0] = jnp.sum(psum_ref[...] * cnt_ref[...]) / (N * N)
        iota1 = lax.broadcasted_iota(jnp.int32, (1, E), 1)
        starts = []
        s = jnp.int32(0)
        for e in range(E):
            ce = jnp.sum(
                jnp.where(iota1 == e, cnt_ref[...], 0.0)).astype(jnp.int32)
            starts.append(s)
            s = s + ((ce + TMM - 1) // TMM) * TMM
        iota_g = lax.broadcasted_iota(jnp.int32, (1, 128), 1) * TMM
        te = jnp.zeros((1, 128), jnp.int32)
        for e in range(1, E):
            te = te + (iota_g >= starts[e]).astype(jnp.int32)
        te_ref[...] = te
        for j in range(NB):
            selj = sel_s[j, :, :]
            destj = rank_s[j, :, :]
            for e in range(1, E):
                destj = destj + jnp.where(selj == e, starts[e], 0)
            dest_ref[j, :, :] = destj


def _routing(alpha2, x3, Wg, bg2, tp2, Wt, bt2):
    return pl.pallas_call(
        _routing_body,
        grid=(NB,),
        in_specs=[
            pl.BlockSpec(memory_space=pltpu.SMEM),
            pl.BlockSpec((TL, B, D), lambda i: (i, 0, 0)),
            pl.BlockSpec((D, E), lambda i: (0, 0)),
            pl.BlockSpec((1, E), lambda i: (0, 0)),
            pl.BlockSpec((1, D), lambda i: (0, 0)),
            pl.BlockSpec((D, E), lambda i: (0, 0)),
            pl.BlockSpec((1, E), lambda i: (0, 0)),
        ],
        out_specs=[
            pl.BlockSpec((TM, D2), lambda i: (i, 0)),
            pl.BlockSpec((NB, 1, TM), lambda i: (0, 0, 0)),
            pl.BlockSpec((1, 128), lambda i: (0, 0)),
            pl.BlockSpec(memory_space=pltpu.SMEM),
        ],
        out_shape=[
            jax.ShapeDtypeStruct((N, D2), jnp.int32),
            jax.ShapeDtypeStruct((NB, 1, TM), jnp.int32),
            jax.ShapeDtypeStruct((1, 128), jnp.int32),
            jax.ShapeDtypeStruct((1, 1), jnp.float32),
        ],
        scratch_shapes=[
            pltpu.VMEM((1, E), jnp.float32),
            pltpu.VMEM((1, E), jnp.float32),
            pltpu.VMEM((NB, 1, TM), jnp.int32),
            pltpu.VMEM((NB, 1, TM), jnp.int32),
        ],
    )(alpha2, x3, Wg, bg2, tp2, Wt, bt2)


_SC_MESH = plsc.VectorSubcoreMesh(core_axis_name="c", subcore_axis_name="s")


@functools.partial(
    pl.kernel, mesh=_SC_MESH,
    out_type=jax.ShapeDtypeStruct((NPAD, D2), jnp.int32),
    scratch_types=[
        pltpu.VMEM((CH,), jnp.int32),
        pltpu.VMEM((CH, D2), jnp.int32),
        pltpu.SemaphoreType.DMA,
    ],
)
def _sc_scatter(x_hbm, dest_hbm, xs_hbm, dest_v, rows_v, sem):
    wid = lax.axis_index("s") * 2 + lax.axis_index("c")
    base = wid * RW
    for k in range(NCH):
        off = base + k * CH
        pltpu.sync_copy(dest_hbm.at[pl.ds(off, CH)], dest_v)
        pltpu.sync_copy(x_hbm.at[pl.ds(off, CH)], rows_v)
        pltpu.async_copy(rows_v, xs_hbm.at[dest_v], sem).wait()


def _gmm_body(te_ref, xs_ref, we_ref, be_ref, ys_ref, wcur_ref, bcur_ref):
    i = pl.program_id(0)
    e = te_ref[i]
    changed = jnp.logical_or(i == 0, te_ref[jnp.maximum(i - 1, 0)] != e)

    @pl.when(changed)
    def _stage_weights():
        wcur_ref[...] = we_ref[pl.ds(e, 1)][0]
        bcur_ref[...] = be_ref[pl.ds(e, 1)][0]

    x = _unpack_bf16(xs_ref[...])
    ys_ref[...] = _pack_bf16(
        jnp.dot(x, wcur_ref[...], preferred_element_type=jnp.float32)
        + bcur_ref[...])


def _gmm(te, xs, We, be):
    grid_spec = pltpu.PrefetchScalarGridSpec(
        num_scalar_prefetch=1,
        grid=(G,),
        in_specs=[
            pl.BlockSpec((TMM, D2), lambda i, te: (i, 0)),
            pl.BlockSpec((E, D, D), lambda i, te: (0, 0, 0)),
            pl.BlockSpec((E, 1, D), lambda i, te: (0, 0, 0)),
        ],
        out_specs=pl.BlockSpec((TMM, D2), lambda i, te: (i, 0)),
        scratch_shapes=[
            pltpu.VMEM((D, D), jnp.bfloat16),
            pltpu.VMEM((1, D), jnp.float32),
        ],
    )
    return pl.pallas_call(
        _gmm_body,
        grid_spec=grid_spec,
        out_shape=jax.ShapeDtypeStruct((NPAD, D2), jnp.int32),
    )(te, xs, We.astype(jnp.bfloat16), be.reshape(E, 1, D))


@functools.partial(
    pl.kernel, mesh=_SC_MESH,
    out_type=jax.ShapeDtypeStruct((N, D2), jnp.int32),
    scratch_types=[
        pltpu.VMEM((CH,), jnp.int32),
        pltpu.VMEM((CH, D2), jnp.int32),
        pltpu.SemaphoreType.DMA,
    ],
)
def _sc_gather(ys_hbm, dest_hbm, out_hbm, dest_v, rows_v, sem):
    wid = lax.axis_index("s") * 2 + lax.axis_index("c")
    base = wid * RW
    for k in range(NCH):
        off = base + k * CH
        pltpu.sync_copy(dest_hbm.at[pl.ds(off, CH)], dest_v)
        pltpu.async_copy(ys_hbm.at[dest_v], rows_v, sem).wait()
        pltpu.sync_copy(rows_v, out_hbm.at[pl.ds(off, CH)])


TMR = 2048


def _relayout_body(ylin_ref, out_ref):
    y = _unpack_bf16(ylin_ref[...]).astype(jnp.float32)
    out_ref[...] = y.reshape(TMR // B, B, D)


def _relayout(ylin):
    return pl.pallas_call(
        _relayout_body,
        grid=(N // TMR,),
        in_specs=[pl.BlockSpec((TMR, D2), lambda i: (i, 0))],
        out_specs=pl.BlockSpec((TMR // B, B, D), lambda i: (i, 0, 0)),
        out_shape=jax.ShapeDtypeStruct((L, B, D), jnp.float32),
    )(ylin)


@jax.jit
def kernel(inputs, task_param, alpha, Wg, bg, Wt, bt, We, be):
    alpha2 = jnp.asarray(alpha, jnp.float32).reshape(1, 1)
    xlin, dest3, te, laux = _routing(
        alpha2, inputs, Wg, bg.reshape(1, E), task_param.reshape(1, D), Wt,
        bt.reshape(1, E))
    dest = dest3.reshape(N)
    xs = _sc_scatter(xlin, dest)
    ys = _gmm(te.reshape(128)[:G], xs, We, be)
    out_lin = _sc_gather(ys, dest)
    return _relayout(out_lin), laux[0, 0]


# speedup vs baseline: 1.1978x; 1.1123x over previous
import functools

import jax
import jax.numpy as jnp
from jax import lax
from jax.experimental import pallas as pl
from jax.experimental.pallas import tpu as pltpu
from jax.experimental.pallas import tpu_sc as plsc

L, B, D, E = 8192, 2, 768, 8
N = L * B
TM = 512
TL = TM // B
NB = N // TM
TMM = 512
G = N // TMM + E
NPAD = G * TMM
NW = 32
RW = N // NW
CH = 128
NCH = RW // CH
D2 = D // 2


def _pack_bf16(x32):
    xb = x32.astype(jnp.bfloat16)
    hi = lax.bitcast_convert_type(xb[:, :D2], jnp.uint16).astype(jnp.int32)
    lo = lax.bitcast_convert_type(xb[:, D2:], jnp.uint16).astype(jnp.int32)
    return (hi << 16) | lo


def _unpack_bf16(xi):
    hi = lax.bitcast_convert_type((xi >> 16).astype(jnp.uint16), jnp.bfloat16)
    lo = lax.bitcast_convert_type((xi & 0xFFFF).astype(jnp.uint16),
                                  jnp.bfloat16)
    return jnp.concatenate([hi, lo], axis=1)


def _routing_body(alpha_ref, x_ref, wg_ref, bg_ref, tp_ref, wt_ref, bt_ref,
                  xlin_ref, dest_ref, te_ref, laux_ref,
                  psum_ref, cnt_ref, sel_s, rank_s):
    i = pl.program_id(0)
    alpha = alpha_ref[0, 0]
    x = x_ref[...].reshape(TM, D)
    xlin_ref[...] = _pack_bf16(x)

    task_logits = (jnp.dot(tp_ref[...], wt_ref[...],
                           preferred_element_type=jnp.float32)
                   + bt_ref[...])
    logits = ((1.0 - alpha)
              * (jnp.dot(x, wg_ref[...], preferred_element_type=jnp.float32)
                 + bg_ref[...])
              + alpha * task_logits)
    logits = jnp.where(jnp.isfinite(logits), logits, 0.0)

    mx = jnp.max(logits, axis=-1, keepdims=True)
    iota_e = lax.broadcasted_iota(jnp.int32, (TM, E), 1)
    sel = jnp.min(jnp.where(logits == mx, iota_e, E), axis=-1,
                  keepdims=True)
    onehot = (iota_e == sel).astype(jnp.float32)

    ex = jnp.exp(logits - mx)
    probs = ex / jnp.sum(ex, axis=-1, keepdims=True)

    @pl.when(i == 0)
    def _init():
        psum_ref[...] = jnp.zeros_like(psum_ref)
        cnt_ref[...] = jnp.zeros_like(cnt_ref)

    lt = (lax.broadcasted_iota(jnp.int32, (TM, TM), 0)
          > lax.broadcasted_iota(jnp.int32, (TM, TM), 1)).astype(jnp.float32)
    ltoh = jnp.dot(lt, onehot, preferred_element_type=jnp.float32)
    rank = jnp.sum(onehot * (cnt_ref[...] + ltoh), axis=-1,
                   keepdims=True)
    sel_s[i, :, :] = sel.astype(jnp.int32).reshape(1, TM)
    rank_s[i, :, :] = rank.astype(jnp.int32).reshape(1, TM)

    psum_ref[...] += jnp.sum(probs, axis=0, keepdims=True)
    cnt_ref[...] += jnp.sum(onehot, axis=0, keepdims=True)

    @pl.when(i == NB - 1)
    def _fin():
        laux_ref[0, 0] = jnp.sum(psum_ref[...] * cnt_ref[...]) / (N * N)
        iota1 = lax.broadcasted_iota(jnp.int32, (1, E), 1)
        starts = []
        s = jnp.int32(0)
        for e in range(E):
            ce = jnp.sum(
                jnp.where(iota1 == e, cnt_ref[...], 0.0)).astype(jnp.int32)
            starts.append(s)
            s = s + ((ce + TMM - 1) // TMM) * TMM
        iota_g = lax.broadcasted_iota(jnp.int32, (1, 128), 1) * TMM
        te = jnp.zeros((1, 128), jnp.int32)
        for e in range(1, E):
            te = te + (iota_g >= starts[e]).astype(jnp.int32)
        te_ref[...] = te
        for j in range(NB):
            selj = sel_s[j, :, :]
            destj = rank_s[j, :, :]
            for e in range(1, E):
                destj = destj + jnp.where(selj == e, starts[e], 0)
            dest_ref[j, :, :] = destj


def _routing(alpha2, x3, Wg, bg2, tp2, Wt, bt2):
    return pl.pallas_call(
        _routing_body,
        grid=(NB,),
        in_specs=[
            pl.BlockSpec(memory_space=pltpu.SMEM),
            pl.BlockSpec((TL, B, D), lambda i: (i, 0, 0)),
            pl.BlockSpec((D, E), lambda i: (0, 0)),
            pl.BlockSpec((1, E), lambda i: (0, 0)),
            pl.BlockSpec((1, D), lambda i: (0, 0)),
            pl.BlockSpec((D, E), lambda i: (0, 0)),
            pl.BlockSpec((1, E), lambda i: (0, 0)),
        ],
        out_specs=[
            pl.BlockSpec((TM, D2), lambda i: (i, 0)),
            pl.BlockSpec((NB, 1, TM), lambda i: (0, 0, 0)),
            pl.BlockSpec((1, 128), lambda i: (0, 0)),
            pl.BlockSpec(memory_space=pltpu.SMEM),
        ],
        out_shape=[
            jax.ShapeDtypeStruct((N, D2), jnp.int32),
            jax.ShapeDtypeStruct((NB, 1, TM), jnp.int32),
            jax.ShapeDtypeStruct((1, 128), jnp.int32),
            jax.ShapeDtypeStruct((1, 1), jnp.float32),
        ],
        scratch_shapes=[
            pltpu.VMEM((1, E), jnp.float32),
            pltpu.VMEM((1, E), jnp.float32),
            pltpu.VMEM((NB, 1, TM), jnp.int32),
            pltpu.VMEM((NB, 1, TM), jnp.int32),
        ],
    )(alpha2, x3, Wg, bg2, tp2, Wt, bt2)


_SC_MESH = plsc.VectorSubcoreMesh(core_axis_name="c", subcore_axis_name="s")


@functools.partial(
    pl.kernel, mesh=_SC_MESH,
    out_type=jax.ShapeDtypeStruct((NPAD, D2), jnp.int32),
    scratch_types=[
        pltpu.VMEM((CH,), jnp.int32),
        pltpu.VMEM((CH, D2), jnp.int32),
        pltpu.SemaphoreType.DMA,
    ],
)
def _sc_scatter(x_hbm, dest_hbm, xs_hbm, dest_v, rows_v, sem):
    wid = lax.axis_index("s") * 2 + lax.axis_index("c")
    base = wid * RW
    for k in range(NCH):
        off = base + k * CH
        pltpu.sync_copy(dest_hbm.at[pl.ds(off, CH)], dest_v)
        pltpu.sync_copy(x_hbm.at[pl.ds(off, CH)], rows_v)
        pltpu.async_copy(rows_v, xs_hbm.at[dest_v], sem).wait()


def _gmm_body(te_ref, xs_ref, we_ref, be_ref, ys_ref, wcur_ref, bcur_ref):
    i = pl.program_id(0)
    e = te_ref[i]
    changed = jnp.logical_or(i == 0, te_ref[jnp.maximum(i - 1, 0)] != e)

    @pl.when(changed)
    def _stage_weights():
        wcur_ref[...] = we_ref[pl.ds(e, 1)][0]
        bcur_ref[...] = be_ref[pl.ds(e, 1)][0]

    x = _unpack_bf16(xs_ref[...])
    ys_ref[...] = _pack_bf16(
        jnp.dot(x, wcur_ref[...], preferred_element_type=jnp.float32)
        + bcur_ref[...])


def _gmm(te, xs, We, be):
    grid_spec = pltpu.PrefetchScalarGridSpec(
        num_scalar_prefetch=1,
        grid=(G,),
        in_specs=[
            pl.BlockSpec((TMM, D2), lambda i, te: (i, 0)),
            pl.BlockSpec((E, D, D), lambda i, te: (0, 0, 0)),
            pl.BlockSpec((E, 1, D), lambda i, te: (0, 0, 0)),
        ],
        out_specs=pl.BlockSpec((TMM, D2), lambda i, te: (i, 0)),
        scratch_shapes=[
            pltpu.VMEM((D, D), jnp.bfloat16),
            pltpu.VMEM((1, D), jnp.float32),
        ],
    )
    return pl.pallas_call(
        _gmm_body,
        grid_spec=grid_spec,
        out_shape=jax.ShapeDtypeStruct((NPAD, D2), jnp.int32),
    )(te, xs, We.astype(jnp.bfloat16), be.reshape(E, 1, D))


@functools.partial(
    pl.kernel, mesh=_SC_MESH,
    out_type=jax.ShapeDtypeStruct((N, D2), jnp.int32),
    scratch_types=[
        pltpu.VMEM((CH,), jnp.int32),
        pltpu.VMEM((CH, D2), jnp.int32),
        pltpu.SemaphoreType.DMA,
    ],
)
def _sc_gather(ys_hbm, dest_hbm, out_hbm, dest_v, rows_v, sem):
    wid = lax.axis_index("s") * 2 + lax.axis_index("c")
    base = wid * RW
    for k in range(NCH):
        off = base + k * CH
        pltpu.sync_copy(dest_hbm.at[pl.ds(off, CH)], dest_v)
        pltpu.async_copy(ys_hbm.at[dest_v], rows_v, sem).wait()
        pltpu.sync_copy(rows_v, out_hbm.at[pl.ds(off, CH)])


TMR = 2048


def _relayout_body(ylin_ref, out_ref):
    y = _unpack_bf16(ylin_ref[...]).astype(jnp.float32)
    out_ref[...] = y.reshape(TMR // B, B, D)


def _relayout(ylin):
    return pl.pallas_call(
        _relayout_body,
        grid=(N // TMR,),
        in_specs=[pl.BlockSpec((TMR, D2), lambda i: (i, 0))],
        out_specs=pl.BlockSpec((TMR // B, B, D), lambda i: (i, 0, 0)),
        out_shape=jax.ShapeDtypeStruct((L, B, D), jnp.float32),
    )(ylin)


@jax.jit
def kernel(inputs, task_param, alpha, Wg, bg, Wt, bt, We, be):
    alpha2 = jnp.asarray(alpha, jnp.float32).reshape(1, 1)
    xlin, dest3, te, laux = _routing(
        alpha2, inputs, Wg, bg.reshape(1, E), task_param.reshape(1, D), Wt,
        bt.reshape(1, E))
    dest = dest3.reshape(N)
    xs = _sc_scatter(xlin, dest)
    ys = _gmm(te.reshape(128)[:G], xs, We, be)
    out_lin = _sc_gather(ys, dest)
    return _relayout(out_lin), laux[0, 0]
